# speedup vs baseline: 3.5245x; 3.5245x over previous
import functools

import numpy as np
import jax
import jax.numpy as jnp
from jax import lax
from jax.experimental import pallas as pl
from jax.experimental.pallas import tpu as pltpu
from jax.experimental.pallas import tpu_sc as plsc

VOCAB = 100000
D = 128
OUT = 128
MAX_LEN = 512
L = 200
B = 1024

NC, NS = 2, 16
NW = NC * NS
TOT = B * L
PER_W = TOT // NW
SEQ_PER_W = PER_W // L
GW = 40
LANES = 16


def _positional_encoding_const():
    pos = np.arange(MAX_LEN, dtype=np.float32)[:, None]
    div = np.exp(np.arange(0, D, 2, dtype=np.float32) * -(np.log(10000.0) / D))
    pe = np.zeros((MAX_LEN, D), dtype=np.float32)
    pe[:, 0::2] = np.sin(pos * div)
    pe[:, 1::2] = np.cos(pos * div)
    return pe[:L]


_PE = _positional_encoding_const()

TBLK = 2000
NTB = VOCAB // TBLK


def _proj_body(tab_ref, w_ref, b_ref, pe_ref, tproj_ref, pep_ref):
    w = w_ref[...]
    acc = lax.dot_general(tab_ref[...], w, (((1,), (1,)), ((), ())),
                          preferred_element_type=jnp.float32)
    tproj_ref[...] = acc + b_ref[...]

    @pl.when(pl.program_id(0) == 0)
    def _():
        pep_ref[...] = lax.dot_general(pe_ref[...], w, (((1,), (1,)), ((), ())),
                                       preferred_element_type=jnp.float32)


def _project(token_table, W, b, pe):
    return pl.pallas_call(
        _proj_body,
        grid=(NTB,),
        in_specs=[
            pl.BlockSpec((TBLK, D), lambda i: (i, 0)),
            pl.BlockSpec((OUT, D), lambda i: (0, 0)),
            pl.BlockSpec((1, OUT), lambda i: (0, 0)),
            pl.BlockSpec((L, D), lambda i: (0, 0)),
        ],
        out_specs=[
            pl.BlockSpec((TBLK, OUT), lambda i: (i, 0)),
            pl.BlockSpec((L, OUT), lambda i: (0, 0)),
        ],
        out_shape=[
            jax.ShapeDtypeStruct((VOCAB, OUT), jnp.float32),
            jax.ShapeDtypeStruct((L, OUT), jnp.float32),
        ],
    )(token_table, W, b.reshape(1, OUT), pe)


def _sc_gather_add(tproj, seqflat, pep):
    mesh = plsc.VectorSubcoreMesh(core_axis_name="c", subcore_axis_name="s")

    @functools.partial(
        pl.kernel,
        out_type=jax.ShapeDtypeStruct((TOT, OUT), jnp.float32),
        mesh=mesh,
        scratch_types=[
            pltpu.VMEM((L,), jnp.int32),
            pltpu.VMEM((L, OUT), jnp.float32),
            pltpu.VMEM((L, OUT), jnp.float32),
            pltpu.SemaphoreType.DMA,
        ],
    )
    def k(tproj_hbm, seq_hbm, pep_hbm, out_hbm, idx_v, rows_v, pe_v, sem):
        wid = lax.axis_index("s") * NC + lax.axis_index("c")
        base = wid * PER_W
        pltpu.sync_copy(pep_hbm, pe_v)

        @pl.loop(0, SEQ_PER_W)
        def _seq(s):
            off = base + s * L
            pltpu.sync_copy(seq_hbm.at[pl.ds(off, L)], idx_v)
            copies = [
                pltpu.async_copy(
                    tproj_hbm.at[idx_v.at[pl.ds(g * GW, GW)]],
                    rows_v.at[pl.ds(g * GW, GW)],
                    sem,
                )
                for g in range(L // GW)
            ]
            for cp in copies:
                cp.wait()

            @pl.loop(0, L)
            def _row(r):
                for c in range(OUT // LANES):
                    sl = (pl.ds(r, 1), pl.ds(c * LANES, LANES))
                    rows_v.at[sl][...] = rows_v.at[sl][...] + pe_v.at[sl][...]

            pltpu.sync_copy(rows_v, out_hbm.at[pl.ds(off, L)])

    return k(tproj, seqflat, pep)


def kernel(sequence, token_table, W, b):
    pe = jnp.asarray(_PE)
    tproj, pep = _project(token_table, W, b, pe)
    seqflat = sequence.reshape(TOT)
    out = _sc_gather_add(tproj, seqflat, pep)
    return out.reshape(B, L, OUT)


# speedup vs baseline: 5.1010x; 1.4473x over previous
import functools

import numpy as np
import jax
import jax.numpy as jnp
from jax import lax
from jax.experimental import pallas as pl
from jax.experimental.pallas import tpu as pltpu
from jax.experimental.pallas import tpu_sc as plsc

VOCAB = 100000
D = 128
OUT = 128
MAX_LEN = 512
L = 200
B = 1024

NC, NS = 2, 16
NW = NC * NS
TOT = B * L
PER_W = TOT // NW
SEQ_PER_W = PER_W // L
GW = 40
LANES = 16


def _positional_encoding_const():
    pos = np.arange(MAX_LEN, dtype=np.float32)[:, None]
    div = np.exp(np.arange(0, D, 2, dtype=np.float32) * -(np.log(10000.0) / D))
    pe = np.zeros((MAX_LEN, D), dtype=np.float32)
    pe[:, 0::2] = np.sin(pos * div)
    pe[:, 1::2] = np.cos(pos * div)
    return pe[:L]


_PE = _positional_encoding_const()

TBLK = 2000
NTB = VOCAB // TBLK


def _proj_body(tab_ref, w_ref, b_ref, pe_ref, tproj_ref, pep_ref):
    w = w_ref[...]
    acc = lax.dot_general(tab_ref[...], w, (((1,), (1,)), ((), ())),
                          preferred_element_type=jnp.float32)
    tproj_ref[...] = acc + b_ref[...]

    @pl.when(pl.program_id(0) == 0)
    def _():
        pep_ref[...] = lax.dot_general(pe_ref[...], w, (((1,), (1,)), ((), ())),
                                       preferred_element_type=jnp.float32)


def _project(token_table, W, b, pe):
    return pl.pallas_call(
        _proj_body,
        grid=(NTB,),
        in_specs=[
            pl.BlockSpec((TBLK, D), lambda i: (i, 0)),
            pl.BlockSpec((OUT, D), lambda i: (0, 0)),
            pl.BlockSpec((1, OUT), lambda i: (0, 0)),
            pl.BlockSpec((L, D), lambda i: (0, 0)),
        ],
        out_specs=[
            pl.BlockSpec((TBLK, OUT), lambda i: (i, 0)),
            pl.BlockSpec((L, OUT), lambda i: (0, 0)),
        ],
        out_shape=[
            jax.ShapeDtypeStruct((VOCAB, OUT), jnp.float32),
            jax.ShapeDtypeStruct((L, OUT), jnp.float32),
        ],
    )(token_table, W, b.reshape(1, OUT), pe)


def _sc_gather_add(tproj, seqflat, pep):
    mesh = plsc.VectorSubcoreMesh(core_axis_name="c", subcore_axis_name="s")

    NSLOT = 3

    @functools.partial(
        pl.kernel,
        out_type=jax.ShapeDtypeStruct((TOT, OUT), jnp.float32),
        mesh=mesh,
        scratch_types=(
            [pltpu.VMEM((L,), jnp.int32) for _ in range(NSLOT)]
            + [pltpu.VMEM((L, OUT), jnp.float32) for _ in range(NSLOT)]
            + [pltpu.VMEM((L, OUT), jnp.float32)]
            + [pltpu.SemaphoreType.DMA for _ in range(2 * NSLOT)]
        ),
    )
    def k(tproj_hbm, seq_hbm, pep_hbm, out_hbm, *scr):
        idx = scr[0:NSLOT]
        rows = scr[NSLOT:2 * NSLOT]
        pe_v = scr[2 * NSLOT]
        gsem = scr[2 * NSLOT + 1:2 * NSLOT + 1 + NSLOT]
        osem = scr[2 * NSLOT + 1 + NSLOT:]
        wid = lax.axis_index("s") * NC + lax.axis_index("c")
        base = wid * PER_W
        pltpu.sync_copy(pep_hbm, pe_v)

        def fire_gathers(s):
            c = s % NSLOT
            off = base + s * L
            pltpu.sync_copy(seq_hbm.at[pl.ds(off, L)], idx[c])
            return [
                pltpu.async_copy(
                    tproj_hbm.at[idx[c].at[pl.ds(g * GW, GW)]],
                    rows[c].at[pl.ds(g * GW, GW)],
                    gsem[c],
                )
                for g in range(L // GW)
            ]

        def add_pe(s):
            c = s % NSLOT

            @pl.loop(0, L)
            def _row(r):
                for col in range(OUT // LANES):
                    sl = (pl.ds(r, 1), pl.ds(col * LANES, LANES))
                    rows[c].at[sl][...] = rows[c].at[sl][...] + pe_v.at[sl][...]

        def fire_out(s):
            c = s % NSLOT
            off = base + s * L
            return pltpu.async_copy(rows[c], out_hbm.at[pl.ds(off, L)], osem[c])

        gh = {}
        oh = {}
        gh[0] = fire_gathers(0)
        for s in range(SEQ_PER_W):
            if s + 1 < SEQ_PER_W:
                if s + 1 - NSLOT >= 0:
                    oh[s + 1 - NSLOT].wait()
                gh[s + 1] = fire_gathers(s + 1)
            for h in gh[s]:
                h.wait()
            add_pe(s)
            oh[s] = fire_out(s)
        for s in range(SEQ_PER_W - NSLOT, SEQ_PER_W):
            if s >= 0:
                oh[s].wait()

    return k(tproj, seqflat, pep)


def kernel(sequence, token_table, W, b):
    pe = jnp.asarray(_PE)
    tproj, pep = _project(token_table, W, b, pe)
    seqflat = sequence.reshape(TOT)
    out = _sc_gather_add(tproj, seqflat, pep)
    return out.reshape(B, L, OUT)


# speedup vs baseline: 5.9027x; 1.1572x over previous
import functools

import numpy as np
import jax
import jax.numpy as jnp
from jax import lax
from jax.experimental import pallas as pl
from jax.experimental.pallas import tpu as pltpu
from jax.experimental.pallas import tpu_sc as plsc

VOCAB = 100000
D = 128
OUT = 128
MAX_LEN = 512
L = 200
B = 1024

NC, NS = 2, 16
NW = NC * NS
TOT = B * L
PER_W = TOT // NW
SEQ_PER_W = PER_W // L
GW = 40
LANES = 16


def _positional_encoding_const():
    pos = np.arange(MAX_LEN, dtype=np.float32)[:, None]
    div = np.exp(np.arange(0, D, 2, dtype=np.float32) * -(np.log(10000.0) / D))
    pe = np.zeros((MAX_LEN, D), dtype=np.float32)
    pe[:, 0::2] = np.sin(pos * div)
    pe[:, 1::2] = np.cos(pos * div)
    return pe[:L]


_PE = _positional_encoding_const()

TBLK = 10000
NTB = VOCAB // TBLK


def _proj_body(tab_ref, w_ref, b_ref, pe_ref, tproj_ref, pep_ref):
    w = w_ref[...]
    acc = lax.dot_general(tab_ref[...], w, (((1,), (1,)), ((), ())),
                          preferred_element_type=jnp.float32)
    tproj_ref[...] = acc + b_ref[...]

    @pl.when(pl.program_id(0) == 0)
    def _():
        pep_ref[...] = lax.dot_general(pe_ref[...], w, (((1,), (1,)), ((), ())),
                                       preferred_element_type=jnp.float32)


def _project(token_table, W, b, pe):
    return pl.pallas_call(
        _proj_body,
        grid=(NTB,),
        in_specs=[
            pl.BlockSpec((TBLK, D), lambda i: (i, 0)),
            pl.BlockSpec((OUT, D), lambda i: (0, 0)),
            pl.BlockSpec((1, OUT), lambda i: (0, 0)),
            pl.BlockSpec((L, D), lambda i: (0, 0)),
        ],
        out_specs=[
            pl.BlockSpec((TBLK, OUT), lambda i: (i, 0)),
            pl.BlockSpec((L, OUT), lambda i: (0, 0)),
        ],
        out_shape=[
            jax.ShapeDtypeStruct((VOCAB, OUT), jnp.float32),
            jax.ShapeDtypeStruct((L, OUT), jnp.float32),
        ],
    )(token_table, W, b.reshape(1, OUT), pe)


def _sc_gather_add(tproj, seqflat, pep):
    mesh = plsc.VectorSubcoreMesh(core_axis_name="c", subcore_axis_name="s")

    NSLOT = 3

    @functools.partial(
        pl.kernel,
        out_type=jax.ShapeDtypeStruct((TOT, OUT), jnp.float32),
        mesh=mesh,
        scratch_types=(
            [pltpu.VMEM((L,), jnp.int32) for _ in range(NSLOT)]
            + [pltpu.VMEM((L, OUT), jnp.float32) for _ in range(NSLOT)]
            + [pltpu.VMEM((L, OUT), jnp.float32)]
            + [pltpu.SemaphoreType.DMA for _ in range(2 * NSLOT)]
        ),
    )
    def k(tproj_hbm, seq_hbm, pep_hbm, out_hbm, *scr):
        idx = scr[0:NSLOT]
        rows = scr[NSLOT:2 * NSLOT]
        pe_v = scr[2 * NSLOT]
        gsem = scr[2 * NSLOT + 1:2 * NSLOT + 1 + NSLOT]
        osem = scr[2 * NSLOT + 1 + NSLOT:]
        wid = lax.axis_index("s") * NC + lax.axis_index("c")
        base = wid * PER_W
        pltpu.sync_copy(pep_hbm, pe_v)

        def fire_gathers(s):
            c = s % NSLOT
            off = base + s * L
            pltpu.sync_copy(seq_hbm.at[pl.ds(off, L)], idx[c])
            return [
                pltpu.async_copy(
                    tproj_hbm.at[idx[c].at[pl.ds(g * GW, GW)]],
                    rows[c].at[pl.ds(g * GW, GW)],
                    gsem[c],
                )
                for g in range(L // GW)
            ]

        def add_pe(s):
            c = s % NSLOT

            @pl.loop(0, L)
            def _row(r):
                for col in range(OUT // LANES):
                    sl = (pl.ds(r, 1), pl.ds(col * LANES, LANES))
                    rows[c].at[sl][...] = rows[c].at[sl][...] + pe_v.at[sl][...]

        def fire_out(s):
            c = s % NSLOT
            off = base + s * L
            return pltpu.async_copy(rows[c], out_hbm.at[pl.ds(off, L)], osem[c])

        gh = {}
        oh = {}
        gh[0] = fire_gathers(0)
        for s in range(SEQ_PER_W):
            if s + 1 < SEQ_PER_W:
                if s + 1 - NSLOT >= 0:
                    oh[s + 1 - NSLOT].wait()
                gh[s + 1] = fire_gathers(s + 1)
            for h in gh[s]:
                h.wait()
            add_pe(s)
            oh[s] = fire_out(s)
        for s in range(SEQ_PER_W - NSLOT, SEQ_PER_W):
            if s >= 0:
                oh[s].wait()

    return k(tproj, seqflat, pep)


def kernel(sequence, token_table, W, b):
    pe = jnp.asarray(_PE)
    tproj, pep = _project(token_table, W, b, pe)
    seqflat = sequence.reshape(TOT)
    out = _sc_gather_add(tproj, seqflat, pep)
    return out.reshape(B, L, OUT)


# speedup vs baseline: 5.9139x; 1.0019x over previous
import functools

import numpy as np
import jax
import jax.numpy as jnp
from jax import lax
from jax.experimental import pallas as pl
from jax.experimental.pallas import tpu as pltpu
from jax.experimental.pallas import tpu_sc as plsc

VOCAB = 100000
D = 128
OUT = 128
MAX_LEN = 512
L = 200
B = 1024

NC, NS = 2, 16
NW = NC * NS
TOT = B * L
PER_W = TOT // NW
SEQ_PER_W = PER_W // L
GW = 40
LANES = 16


def _positional_encoding_const():
    pos = np.arange(MAX_LEN, dtype=np.float32)[:, None]
    div = np.exp(np.arange(0, D, 2, dtype=np.float32) * -(np.log(10000.0) / D))
    pe = np.zeros((MAX_LEN, D), dtype=np.float32)
    pe[:, 0::2] = np.sin(pos * div)
    pe[:, 1::2] = np.cos(pos * div)
    return pe[:L]


_PE = _positional_encoding_const()

TBLK = 10000
NTB = VOCAB // TBLK


def _proj_body(tab_ref, w_ref, b_ref, pe_ref, tproj_ref, pep_ref):
    w = w_ref[...]
    acc = lax.dot_general(tab_ref[...], w, (((1,), (1,)), ((), ())),
                          preferred_element_type=jnp.float32)
    tproj_ref[...] = acc + b_ref[...]

    @pl.when(pl.program_id(0) == 0)
    def _():
        pep_ref[...] = lax.dot_general(pe_ref[...], w, (((1,), (1,)), ((), ())),
                                       preferred_element_type=jnp.float32)


def _project(token_table, W, b, pe):
    return pl.pallas_call(
        _proj_body,
        grid=(NTB,),
        in_specs=[
            pl.BlockSpec((TBLK, D), lambda i: (i, 0)),
            pl.BlockSpec((OUT, D), lambda i: (0, 0)),
            pl.BlockSpec((1, OUT), lambda i: (0, 0)),
            pl.BlockSpec((L, D), lambda i: (0, 0)),
        ],
        out_specs=[
            pl.BlockSpec((TBLK, OUT), lambda i: (i, 0)),
            pl.BlockSpec((L, OUT), lambda i: (0, 0)),
        ],
        out_shape=[
            jax.ShapeDtypeStruct((VOCAB, OUT), jnp.float32),
            jax.ShapeDtypeStruct((L, OUT), jnp.float32),
        ],
    )(token_table, W, b.reshape(1, OUT), pe)


def _sc_gather_add(tproj, seqflat, pep):
    mesh = plsc.VectorSubcoreMesh(core_axis_name="c", subcore_axis_name="s")

    NSLOT = 4

    @functools.partial(
        pl.kernel,
        out_type=jax.ShapeDtypeStruct((TOT, OUT), jnp.float32),
        mesh=mesh,
        scratch_types=(
            [pltpu.VMEM((L,), jnp.int32) for _ in range(NSLOT)]
            + [pltpu.VMEM((L, OUT), jnp.float32) for _ in range(NSLOT)]
            + [pltpu.VMEM((L, OUT), jnp.float32)]
            + [pltpu.SemaphoreType.DMA for _ in range(2 * NSLOT)]
        ),
    )
    def k(tproj_hbm, seq_hbm, pep_hbm, out_hbm, *scr):
        idx = scr[0:NSLOT]
        rows = scr[NSLOT:2 * NSLOT]
        pe_v = scr[2 * NSLOT]
        gsem = scr[2 * NSLOT + 1:2 * NSLOT + 1 + NSLOT]
        osem = scr[2 * NSLOT + 1 + NSLOT:]
        wid = lax.axis_index("s") * NC + lax.axis_index("c")
        base = wid * PER_W
        pltpu.sync_copy(pep_hbm, pe_v)

        def fire_gathers(s):
            c = s % NSLOT
            off = base + s * L
            pltpu.sync_copy(seq_hbm.at[pl.ds(off, L)], idx[c])
            return [
                pltpu.async_copy(
                    tproj_hbm.at[idx[c].at[pl.ds(g * GW, GW)]],
                    rows[c].at[pl.ds(g * GW, GW)],
                    gsem[c],
                )
                for g in range(L // GW)
            ]

        def add_pe(s):
            c = s % NSLOT

            @pl.loop(0, L)
            def _row(r):
                for col in range(OUT // LANES):
                    sl = (pl.ds(r, 1), pl.ds(col * LANES, LANES))
                    rows[c].at[sl][...] = rows[c].at[sl][...] + pe_v.at[sl][...]

        def fire_out(s):
            c = s % NSLOT
            off = base + s * L
            return pltpu.async_copy(rows[c], out_hbm.at[pl.ds(off, L)], osem[c])

        gh = {}
        oh = {}
        gh[0] = fire_gathers(0)
        gh[1] = fire_gathers(1)
        for s in range(SEQ_PER_W):
            if s + 2 < SEQ_PER_W:
                if s + 2 - NSLOT >= 0:
                    oh[s + 2 - NSLOT].wait()
                gh[s + 2] = fire_gathers(s + 2)
            for h in gh[s]:
                h.wait()
            add_pe(s)
            oh[s] = fire_out(s)
        for s in range(SEQ_PER_W - NSLOT, SEQ_PER_W):
            oh[s].wait()

    return k(tproj, seqflat, pep)


def kernel(sequence, token_table, W, b):
    pe = jnp.asarray(_PE)
    tproj, pep = _project(token_table, W, b, pe)
    seqflat = sequence.reshape(TOT)
    out = _sc_gather_add(tproj, seqflat, pep)
    return out.reshape(B, L, OUT)


# speedup vs baseline: 6.2804x; 1.0620x over previous
import functools

import numpy as np
import jax
import jax.numpy as jnp
from jax import lax
from jax.experimental import pallas as pl
from jax.experimental.pallas import tpu as pltpu
from jax.experimental.pallas import tpu_sc as plsc

VOCAB = 100000
D = 128
OUT = 128
MAX_LEN = 512
L = 200
B = 1024

NC, NS = 2, 16
NW = NC * NS
TOT = B * L
PER_W = TOT // NW
SEQ_PER_W = PER_W // L
GW = 40
LANES = 16


def _positional_encoding_const():
    pos = np.arange(MAX_LEN, dtype=np.float32)[:, None]
    div = np.exp(np.arange(0, D, 2, dtype=np.float32) * -(np.log(10000.0) / D))
    pe = np.zeros((MAX_LEN, D), dtype=np.float32)
    pe[:, 0::2] = np.sin(pos * div)
    pe[:, 1::2] = np.cos(pos * div)
    return pe[:L]


_PE = _positional_encoding_const()

TBLK = 10000
NTB = VOCAB // TBLK


def _proj_body(tab_ref, w_ref, b_ref, pe_ref, tproj_ref, pep_ref):
    w = w_ref[...]
    acc = lax.dot_general(tab_ref[...], w, (((1,), (1,)), ((), ())),
                          preferred_element_type=jnp.float32)
    tproj_ref[...] = acc + b_ref[...]

    @pl.when(pl.program_id(0) == 0)
    def _():
        pep_ref[...] = lax.dot_general(pe_ref[...], w, (((1,), (1,)), ((), ())),
                                       preferred_element_type=jnp.float32)


def _project(token_table, W, b, pe):
    return pl.pallas_call(
        _proj_body,
        grid=(NTB,),
        in_specs=[
            pl.BlockSpec((TBLK, D), lambda i: (i, 0)),
            pl.BlockSpec((OUT, D), lambda i: (0, 0)),
            pl.BlockSpec((1, OUT), lambda i: (0, 0)),
            pl.BlockSpec((L, D), lambda i: (0, 0)),
        ],
        out_specs=[
            pl.BlockSpec((TBLK, OUT), lambda i: (i, 0)),
            pl.BlockSpec((L, OUT), lambda i: (0, 0)),
        ],
        out_shape=[
            jax.ShapeDtypeStruct((VOCAB, OUT), jnp.float32),
            jax.ShapeDtypeStruct((L, OUT), jnp.float32),
        ],
    )(token_table, W, b.reshape(1, OUT), pe)


def _sc_gather_add(tproj, seqflat, pep):
    mesh = plsc.VectorSubcoreMesh(core_axis_name="c", subcore_axis_name="s")

    NSLOT = 3

    @functools.partial(
        pl.kernel,
        out_type=jax.ShapeDtypeStruct((TOT, OUT), jnp.float32),
        mesh=mesh,
        scratch_types=(
            [pltpu.VMEM((PER_W,), jnp.int32)]
            + [pltpu.VMEM((L, OUT), jnp.float32) for _ in range(NSLOT)]
            + [pltpu.VMEM((L, OUT), jnp.float32)]
            + [pltpu.SemaphoreType.DMA for _ in range(2 * NSLOT)]
        ),
    )
    def k(tproj_hbm, seq_hbm, pep_hbm, out_hbm, *scr):
        idx_all = scr[0]
        rows = scr[1:1 + NSLOT]
        pe_v = scr[1 + NSLOT]
        gsem = scr[2 + NSLOT:2 + 2 * NSLOT]
        osem = scr[2 + 2 * NSLOT:]
        wid = lax.axis_index("s") * NC + lax.axis_index("c")
        base = wid * PER_W
        pltpu.sync_copy(seq_hbm.at[pl.ds(base, PER_W)], idx_all)
        pltpu.sync_copy(pep_hbm, pe_v)

        def fire_gathers(s):
            c = s % NSLOT
            return [
                pltpu.async_copy(
                    tproj_hbm.at[idx_all.at[pl.ds(s * L + g * GW, GW)]],
                    rows[c].at[pl.ds(g * GW, GW)],
                    gsem[c],
                )
                for g in range(L // GW)
            ]

        def add_pe(s):
            c = s % NSLOT

            @pl.loop(0, L)
            def _row(r):
                for col in range(OUT // LANES):
                    sl = (pl.ds(r, 1), pl.ds(col * LANES, LANES))
                    rows[c].at[sl][...] = rows[c].at[sl][...] + pe_v.at[sl][...]

        def fire_out(s):
            c = s % NSLOT
            off = base + s * L
            return pltpu.async_copy(rows[c], out_hbm.at[pl.ds(off, L)], osem[c])

        gh = {}
        oh = {}
        gh[0] = fire_gathers(0)
        for s in range(SEQ_PER_W):
            if s + 1 < SEQ_PER_W:
                if s + 1 - NSLOT >= 0:
                    oh[s + 1 - NSLOT].wait()
                gh[s + 1] = fire_gathers(s + 1)
            for h in gh[s]:
                h.wait()
            add_pe(s)
            oh[s] = fire_out(s)
        for s in range(SEQ_PER_W - NSLOT, SEQ_PER_W):
            oh[s].wait()

    return k(tproj, seqflat, pep)


def kernel(sequence, token_table, W, b):
    pe = jnp.asarray(_PE)
    tproj, pep = _project(token_table, W, b, pe)
    seqflat = sequence.reshape(TOT)
    out = _sc_gather_add(tproj, seqflat, pep)
    return out.reshape(B, L, OUT)


# speedup vs baseline: 6.3249x; 1.0071x over previous
import functools

import numpy as np
import jax
import jax.numpy as jnp
from jax import lax
from jax.experimental import pallas as pl
from jax.experimental.pallas import tpu as pltpu
from jax.experimental.pallas import tpu_sc as plsc

VOCAB = 100000
D = 128
OUT = 128
MAX_LEN = 512
L = 200
B = 1024

NC, NS = 2, 16
NW = NC * NS
TOT = B * L
PER_W = TOT // NW
SEQ_PER_W = PER_W // L
GW = 40
LANES = 16


def _positional_encoding_const():
    pos = np.arange(MAX_LEN, dtype=np.float32)[:, None]
    div = np.exp(np.arange(0, D, 2, dtype=np.float32) * -(np.log(10000.0) / D))
    pe = np.zeros((MAX_LEN, D), dtype=np.float32)
    pe[:, 0::2] = np.sin(pos * div)
    pe[:, 1::2] = np.cos(pos * div)
    return pe[:L]


_PE = _positional_encoding_const()

TBLK = 10000
NTB = VOCAB // TBLK


def _proj_body(tab_ref, w_ref, b_ref, pe_ref, tproj_ref, pep_ref):
    w = w_ref[...]
    acc = lax.dot_general(tab_ref[...], w, (((1,), (1,)), ((), ())),
                          preferred_element_type=jnp.float32)
    tproj_ref[...] = acc + b_ref[...]

    @pl.when(pl.program_id(0) == 0)
    def _():
        pep_ref[...] = lax.dot_general(pe_ref[...], w, (((1,), (1,)), ((), ())),
                                       preferred_element_type=jnp.float32)


def _project(token_table, W, b, pe):
    return pl.pallas_call(
        _proj_body,
        grid=(NTB,),
        in_specs=[
            pl.BlockSpec((TBLK, D), lambda i: (i, 0)),
            pl.BlockSpec((OUT, D), lambda i: (0, 0)),
            pl.BlockSpec((1, OUT), lambda i: (0, 0)),
            pl.BlockSpec((L, D), lambda i: (0, 0)),
        ],
        out_specs=[
            pl.BlockSpec((TBLK, OUT), lambda i: (i, 0)),
            pl.BlockSpec((L, OUT), lambda i: (0, 0)),
        ],
        out_shape=[
            jax.ShapeDtypeStruct((VOCAB, OUT), jnp.float32),
            jax.ShapeDtypeStruct((L, OUT), jnp.float32),
        ],
    )(token_table, W, b.reshape(1, OUT), pe)


def _sc_gather_add(tproj, seqflat, pep):
    mesh = plsc.VectorSubcoreMesh(core_axis_name="c", subcore_axis_name="s")

    NSLOT = 3

    @functools.partial(
        pl.kernel,
        out_type=jax.ShapeDtypeStruct((TOT, OUT), jnp.float32),
        mesh=mesh,
        scratch_types=(
            [pltpu.VMEM((PER_W,), jnp.int32)]
            + [pltpu.VMEM((L, OUT), jnp.float32) for _ in range(NSLOT)]
            + [pltpu.VMEM((L, OUT), jnp.float32)]
            + [pltpu.SemaphoreType.DMA for _ in range(2 * NSLOT)]
        ),
    )
    def k(tproj_hbm, seq_hbm, pep_hbm, out_hbm, *scr):
        idx_all = scr[0]
        rows = scr[1:1 + NSLOT]
        pe_v = scr[1 + NSLOT]
        gsem = scr[2 + NSLOT:2 + 2 * NSLOT]
        osem = scr[2 + 2 * NSLOT:]
        wid = lax.axis_index("s") * NC + lax.axis_index("c")
        base = wid * PER_W
        pltpu.sync_copy(seq_hbm.at[pl.ds(base, PER_W)], idx_all)
        pltpu.sync_copy(pep_hbm, pe_v)

        def fire_gathers(s):
            c = s % NSLOT
            return [
                pltpu.async_copy(
                    tproj_hbm.at[idx_all.at[pl.ds(s * L + o, n)]],
                    rows[c].at[pl.ds(o, n)],
                    gsem[c],
                )
                for o, n in ((0, 104), (104, 96))
            ]

        def add_pe(s):
            c = s % NSLOT

            @pl.loop(0, L)
            def _row(r):
                for col in range(OUT // LANES):
                    sl = (pl.ds(r, 1), pl.ds(col * LANES, LANES))
                    rows[c].at[sl][...] = rows[c].at[sl][...] + pe_v.at[sl][...]

        def fire_out(s):
            c = s % NSLOT
            off = base + s * L
            return pltpu.async_copy(rows[c], out_hbm.at[pl.ds(off, L)], osem[c])

        gh = {}
        oh = {}
        gh[0] = fire_gathers(0)
        for s in range(SEQ_PER_W):
            if s + 1 < SEQ_PER_W:
                if s + 1 - NSLOT >= 0:
                    oh[s + 1 - NSLOT].wait()
                gh[s + 1] = fire_gathers(s + 1)
            for h in gh[s]:
                h.wait()
            add_pe(s)
            oh[s] = fire_out(s)
        for s in range(SEQ_PER_W - NSLOT, SEQ_PER_W):
            oh[s].wait()

    return k(tproj, seqflat, pep)


def kernel(sequence, token_table, W, b):
    pe = jnp.asarray(_PE)
    tproj, pep = _project(token_table, W, b, pe)
    seqflat = sequence.reshape(TOT)
    out = _sc_gather_add(tproj, seqflat, pep)
    return out.reshape(B, L, OUT)


# speedup vs baseline: 6.3511x; 1.0041x over previous
import functools

import numpy as np
import jax
import jax.numpy as jnp
from jax import lax
from jax.experimental import pallas as pl
from jax.experimental.pallas import tpu as pltpu
from jax.experimental.pallas import tpu_sc as plsc

VOCAB = 100000
D = 128
OUT = 128
MAX_LEN = 512
L = 200
B = 1024

NC, NS = 2, 16
NW = NC * NS
TOT = B * L
PER_W = TOT // NW
SEQ_PER_W = PER_W // L
GW = 40
LANES = 16


def _positional_encoding_const():
    pos = np.arange(MAX_LEN, dtype=np.float32)[:, None]
    div = np.exp(np.arange(0, D, 2, dtype=np.float32) * -(np.log(10000.0) / D))
    pe = np.zeros((MAX_LEN, D), dtype=np.float32)
    pe[:, 0::2] = np.sin(pos * div)
    pe[:, 1::2] = np.cos(pos * div)
    return pe[:L]


_PE = _positional_encoding_const()

TBLK = 20000
NTB = VOCAB // TBLK


def _proj_body(tab_ref, w_ref, b_ref, pe_ref, tproj_ref, pep_ref):
    w = w_ref[...]
    acc = lax.dot_general(tab_ref[...], w, (((1,), (1,)), ((), ())),
                          preferred_element_type=jnp.float32)
    tproj_ref[...] = acc + b_ref[...]

    @pl.when(pl.program_id(0) == 0)
    def _():
        pep_ref[...] = lax.dot_general(pe_ref[...], w, (((1,), (1,)), ((), ())),
                                       preferred_element_type=jnp.float32)


def _project(token_table, W, b, pe):
    return pl.pallas_call(
        _proj_body,
        grid=(NTB,),
        in_specs=[
            pl.BlockSpec((TBLK, D), lambda i: (i, 0)),
            pl.BlockSpec((OUT, D), lambda i: (0, 0)),
            pl.BlockSpec((1, OUT), lambda i: (0, 0)),
            pl.BlockSpec((L, D), lambda i: (0, 0)),
        ],
        out_specs=[
            pl.BlockSpec((TBLK, OUT), lambda i: (i, 0)),
            pl.BlockSpec((L, OUT), lambda i: (0, 0)),
        ],
        out_shape=[
            jax.ShapeDtypeStruct((VOCAB, OUT), jnp.float32),
            jax.ShapeDtypeStruct((L, OUT), jnp.float32),
        ],
    )(token_table, W, b.reshape(1, OUT), pe)


def _sc_gather_add(tproj, seqflat, pep):
    mesh = plsc.VectorSubcoreMesh(core_axis_name="c", subcore_axis_name="s")

    NSLOT = 3

    @functools.partial(
        pl.kernel,
        out_type=jax.ShapeDtypeStruct((TOT, OUT), jnp.float32),
        mesh=mesh,
        scratch_types=(
            [pltpu.VMEM((PER_W,), jnp.int32)]
            + [pltpu.VMEM((L, OUT), jnp.float32) for _ in range(NSLOT)]
            + [pltpu.VMEM((L, OUT), jnp.float32)]
            + [pltpu.SemaphoreType.DMA for _ in range(3 * NSLOT)]
        ),
    )
    def k(tproj_hbm, seq_hbm, pep_hbm, out_hbm, *scr):
        idx_all = scr[0]
        rows = scr[1:1 + NSLOT]
        pe_v = scr[1 + NSLOT]
        gsem = scr[2 + NSLOT:2 + 3 * NSLOT]
        osem = scr[2 + 3 * NSLOT:]
        wid = lax.axis_index("s") * NC + lax.axis_index("c")
        base = wid * PER_W
        pltpu.sync_copy(seq_hbm.at[pl.ds(base, PER_W)], idx_all)
        pltpu.sync_copy(pep_hbm, pe_v)

        WINDOWS = ((0, 104), (104, 96))

        def fire_gathers(s):
            c = s % NSLOT
            return [
                pltpu.async_copy(
                    tproj_hbm.at[idx_all.at[pl.ds(s * L + o, n)]],
                    rows[c].at[pl.ds(o, n)],
                    gsem[2 * c + w],
                )
                for w, (o, n) in enumerate(WINDOWS)
            ]

        def add_pe(s, lo, n):
            c = s % NSLOT

            @pl.loop(lo, lo + n)
            def _row(r):
                for col in range(OUT // LANES):
                    sl = (pl.ds(r, 1), pl.ds(col * LANES, LANES))
                    rows[c].at[sl][...] = rows[c].at[sl][...] + pe_v.at[sl][...]

        def fire_out(s):
            c = s % NSLOT
            off = base + s * L
            return pltpu.async_copy(rows[c], out_hbm.at[pl.ds(off, L)], osem[c])

        gh = {}
        oh = {}
        gh[0] = fire_gathers(0)
        for s in range(SEQ_PER_W):
            if s + 1 < SEQ_PER_W:
                if s + 1 - NSLOT >= 0:
                    oh[s + 1 - NSLOT].wait()
                gh[s + 1] = fire_gathers(s + 1)
            for h, (o, n) in zip(gh[s], WINDOWS):
                h.wait()
                add_pe(s, o, n)
            oh[s] = fire_out(s)
        for s in range(SEQ_PER_W - NSLOT, SEQ_PER_W):
            oh[s].wait()

    return k(tproj, seqflat, pep)


def kernel(sequence, token_table, W, b):
    pe = jnp.asarray(_PE)
    tproj, pep = _project(token_table, W, b, pe)
    seqflat = sequence.reshape(TOT)
    out = _sc_gather_add(tproj, seqflat, pep)
    return out.reshape(B, L, OUT)


# speedup vs baseline: 6.4525x; 1.0160x over previous
import functools

import numpy as np
import jax
import jax.numpy as jnp
from jax import lax
from jax.experimental import pallas as pl
from jax.experimental.pallas import tpu as pltpu
from jax.experimental.pallas import tpu_sc as plsc

VOCAB = 100000
D = 128
OUT = 128
MAX_LEN = 512
L = 200
B = 1024

NC, NS = 2, 16
NW = NC * NS
TOT = B * L
PER_W = TOT // NW
SEQ_PER_W = PER_W // L
GW = 40
LANES = 16


def _positional_encoding_const():
    pos = np.arange(MAX_LEN, dtype=np.float32)[:, None]
    div = np.exp(np.arange(0, D, 2, dtype=np.float32) * -(np.log(10000.0) / D))
    pe = np.zeros((MAX_LEN, D), dtype=np.float32)
    pe[:, 0::2] = np.sin(pos * div)
    pe[:, 1::2] = np.cos(pos * div)
    return pe[:L]


_PE = _positional_encoding_const()

TBLK = 20000
NTB = VOCAB // TBLK


def _proj_body(tab_ref, w_ref, b_ref, pe_ref, tproj_ref, pep_ref):
    w = w_ref[...]
    acc = lax.dot_general(tab_ref[...], w, (((1,), (1,)), ((), ())),
                          preferred_element_type=jnp.float32)
    tproj_ref[...] = acc + b_ref[...]

    @pl.when(pl.program_id(0) == 0)
    def _():
        pep_ref[...] = lax.dot_general(pe_ref[...], w, (((1,), (1,)), ((), ())),
                                       preferred_element_type=jnp.float32)


def _project(token_table, W, b, pe):
    return pl.pallas_call(
        _proj_body,
        grid=(NTB,),
        in_specs=[
            pl.BlockSpec((TBLK, D), lambda i: (i, 0)),
            pl.BlockSpec((OUT, D), lambda i: (0, 0)),
            pl.BlockSpec((1, OUT), lambda i: (0, 0)),
            pl.BlockSpec((L, D), lambda i: (0, 0)),
        ],
        out_specs=[
            pl.BlockSpec((TBLK, OUT), lambda i: (i, 0)),
            pl.BlockSpec((L, OUT), lambda i: (0, 0)),
        ],
        out_shape=[
            jax.ShapeDtypeStruct((VOCAB, OUT), jnp.float32),
            jax.ShapeDtypeStruct((L, OUT), jnp.float32),
        ],
    )(token_table, W, b.reshape(1, OUT), pe)


def _sc_gather_add(tproj, seqflat, pep):
    mesh = plsc.VectorSubcoreMesh(core_axis_name="c", subcore_axis_name="s")

    NSLOT = 3

    @functools.partial(
        pl.kernel,
        out_type=jax.ShapeDtypeStruct((TOT, OUT), jnp.float32),
        mesh=mesh,
        scratch_types=(
            [pltpu.VMEM((PER_W,), jnp.int32)]
            + [pltpu.VMEM((L, OUT), jnp.float32) for _ in range(NSLOT)]
            + [pltpu.VMEM((L, OUT), jnp.float32)]
            + [pltpu.SemaphoreType.DMA for _ in range(3 * NSLOT + 2)]
        ),
    )
    def k(tproj_hbm, seq_hbm, pep_hbm, out_hbm, *scr):
        idx_all = scr[0]
        rows = scr[1:1 + NSLOT]
        pe_v = scr[1 + NSLOT]
        gsem = scr[2 + NSLOT:2 + 3 * NSLOT]
        osem = scr[2 + 3 * NSLOT:2 + 4 * NSLOT]
        ssem_i, ssem_p = scr[2 + 4 * NSLOT:]
        wid = lax.axis_index("s") * NC + lax.axis_index("c")
        base = wid * PER_W
        h_idx = pltpu.async_copy(seq_hbm.at[pl.ds(base, PER_W)], idx_all, ssem_i)
        h_pe = pltpu.async_copy(pep_hbm, pe_v, ssem_p)
        h_idx.wait()

        WINDOWS = ((0, 104), (104, 96))

        def fire_gathers(s):
            c = s % NSLOT
            return [
                pltpu.async_copy(
                    tproj_hbm.at[idx_all.at[pl.ds(s * L + o, n)]],
                    rows[c].at[pl.ds(o, n)],
                    gsem[2 * c + w],
                )
                for w, (o, n) in enumerate(WINDOWS)
            ]

        def add_pe(s, lo, n):
            c = s % NSLOT

            @pl.loop(lo, lo + n)
            def _row(r):
                for col in range(OUT // LANES):
                    sl = (pl.ds(r, 1), pl.ds(col * LANES, LANES))
                    rows[c].at[sl][...] = rows[c].at[sl][...] + pe_v.at[sl][...]

        def fire_out(s):
            c = s % NSLOT
            off = base + s * L
            return pltpu.async_copy(rows[c], out_hbm.at[pl.ds(off, L)], osem[c])

        gh = {}
        oh = {}
        gh[0] = fire_gathers(0)
        h_pe.wait()
        for s in range(SEQ_PER_W):
            if s + 1 < SEQ_PER_W:
                if s + 1 - NSLOT >= 0:
                    oh[s + 1 - NSLOT].wait()
                gh[s + 1] = fire_gathers(s + 1)
            for h, (o, n) in zip(gh[s], WINDOWS):
                h.wait()
                add_pe(s, o, n)
            oh[s] = fire_out(s)
        for s in range(SEQ_PER_W - NSLOT, SEQ_PER_W):
            oh[s].wait()

    return k(tproj, seqflat, pep)


def kernel(sequence, token_table, W, b):
    pe = jnp.asarray(_PE)
    tproj, pep = _project(token_table, W, b, pe)
    seqflat = sequence.reshape(TOT)
    out = _sc_gather_add(tproj, seqflat, pep)
    return out.reshape(B, L, OUT)
